# baseline (speedup 1.0000x reference)
import functools

import jax
import jax.numpy as jnp
from jax import lax
from jax.experimental import pallas as pl
from jax.experimental.pallas import tpu as pltpu
from jax.experimental.pallas import tpu_sc as plsc

VOCAB = 1000000
EMBED = 64
B = 16384
K = 20

NC = 2
NS = 16
L = 16
NW = NC * NS
BPW = B // NW
SB = 32
NSUB = BPW // SB
CTX_SB = SB * K


def _widen(v):
    lo = plsc.bitcast(v << 16, jnp.float32)
    hi = plsc.bitcast(v & jnp.int32(-65536), jnp.float32)
    return lo, hi


def _sc_body(ctr_idx_hbm, ctx_idx_hbm, ctr_tab, ctx_tab, out_hbm,
             ctr_idx_v, ctx_idx_v, ctr_rows, ctx_buf0, ctx_buf1, out_v,
             sem_c, sem0, sem1):
    wid = lax.axis_index("s") * NC + lax.axis_index("c")
    b0 = wid * BPW

    pltpu.sync_copy(ctr_idx_hbm.at[pl.ds(b0, BPW)], ctr_idx_v)
    pltpu.sync_copy(ctx_idx_hbm.at[pl.ds(b0 * K, BPW * K)], ctx_idx_v)

    ctr_cp = pltpu.async_copy(ctr_tab.at[ctr_idx_v], ctr_rows, sem_c)
    bufs = (ctx_buf0, ctx_buf1)
    sems = (sem0, sem1)
    cps = [None, None]
    cps[0] = pltpu.async_copy(
        ctx_tab.at[ctx_idx_v.at[pl.ds(0, CTX_SB)]], ctx_buf0, sem0)
    ctr_cp.wait()

    lanes = lax.iota(jnp.int32, L)
    zero = jnp.zeros((L,), jnp.float32)

    for s in range(NSUB):
        cur = s % 2
        if s + 1 < NSUB:
            nxt = (s + 1) % 2
            cps[nxt] = pltpu.async_copy(
                ctx_tab.at[ctx_idx_v.at[pl.ds((s + 1) * CTX_SB, CTX_SB)]],
                bufs[nxt], sems[nxt])
        cps[cur].wait()
        buf_i32 = bufs[cur]

        for g in range(SB // L):
            row_ctr = s * SB + g * L + lanes
            rowbase = (g * L + lanes) * K

            def e_body(ec, accs, row_ctr=row_ctr, rowbase=rowbase,
                       buf_i32=buf_i32):
                col = jnp.full((L,), ec, jnp.int32)
                c0, c1 = _widen(plsc.load_gather(ctr_rows, [row_ctr, col]))
                new = []
                for k in range(K):
                    x0, x1 = _widen(
                        plsc.load_gather(buf_i32, [rowbase + k, col]))
                    new.append(accs[k] + c0 * x0 + c1 * x1)
                return tuple(new)

            accs = lax.fori_loop(0, EMBED // 2, e_body, (zero,) * K)
            for k in range(K):
                plsc.store_scatter(out_v, [rowbase + k], accs[k])

        pltpu.sync_copy(out_v, out_hbm.at[pl.ds((b0 + s * SB) * K, CTX_SB)])


_mesh = plsc.VectorSubcoreMesh(core_axis_name="c", subcore_axis_name="s")

_sc_kernel = functools.partial(
    pl.kernel,
    out_type=jax.ShapeDtypeStruct((B * K,), jnp.float32),
    mesh=_mesh,
    scratch_types=[
        pltpu.VMEM((BPW,), jnp.int32),
        pltpu.VMEM((BPW * K,), jnp.int32),
        pltpu.VMEM((BPW, EMBED // 2), jnp.int32),
        pltpu.VMEM((CTX_SB, EMBED // 2), jnp.int32),
        pltpu.VMEM((CTX_SB, EMBED // 2), jnp.int32),
        pltpu.VMEM((CTX_SB,), jnp.float32),
        pltpu.SemaphoreType.DMA,
        pltpu.SemaphoreType.DMA,
        pltpu.SemaphoreType.DMA,
    ],
    compiler_params=pltpu.CompilerParams(
        use_tc_tiling_on_sc=False, needs_layout_passes=False),
)(_sc_body)


def _pack_bf16(table):
    b = table.astype(jnp.bfloat16).reshape(VOCAB, EMBED // 2, 2)
    return lax.bitcast_convert_type(b, jnp.int32)


def kernel(centers, context_negatives, center_table, context_table):
    ctr_idx = centers.reshape(B).astype(jnp.int32)
    ctx_idx = context_negatives.reshape(B * K).astype(jnp.int32)
    out = _sc_kernel(ctr_idx, ctx_idx,
                     _pack_bf16(center_table), _pack_bf16(context_table))
    return out.reshape(B, K)


# speedup vs baseline: 2.2713x; 2.2713x over previous
import functools

import jax
import jax.numpy as jnp
from jax import lax
from jax.experimental import pallas as pl
from jax.experimental.pallas import tpu as pltpu
from jax.experimental.pallas import tpu_sc as plsc

VOCAB = 1000000
EMBED = 64
B = 16384
K = 20

NC = 2
NS = 16
L = 16
NW = NC * NS
BPW = B // NW
SB = 16
NSUB = BPW // SB
CTX_SB = SB * K
NBUF = 4
NSPLIT = 2


def _sc_body(ctr_idx_hbm, ctx_idx_hbm, ctr_tab, ctx_tab, out_hbm,
             ctr_idx_v, ctx_idx_v, ctr_rows, bufs, out_v, sem_c, sems):
    wid = lax.axis_index("s") * NC + lax.axis_index("c")
    b0 = wid * BPW

    pltpu.sync_copy(ctr_idx_hbm.at[pl.ds(b0, BPW)], ctr_idx_v)
    pltpu.sync_copy(ctx_idx_hbm.at[pl.ds(b0 * K, BPW * K)], ctx_idx_v)

    def fire(s, buf, sem_pair):
        half = CTX_SB // NSPLIT
        return [
            pltpu.async_copy(
                ctx_tab.at[ctx_idx_v.at[pl.ds(s * CTX_SB + j * half, half)]],
                buf.at[pl.ds(j * half, half)],
                sem_pair[j],
            )
            for j in range(NSPLIT)
        ]

    ctr_cp = pltpu.async_copy(ctr_tab.at[ctr_idx_v], ctr_rows, sem_c)
    cps = [None] * NBUF
    for s in range(NBUF - 1):
        cps[s] = fire(s, bufs[s], sems[s])
    ctr_cp.wait()

    lanes = lax.iota(jnp.int32, L)
    zero = jnp.zeros((L,), jnp.float32)

    for s in range(NSUB):
        cur = s % NBUF
        if s + NBUF - 1 < NSUB:
            nf = (s + NBUF - 1) % NBUF
            cps[nf] = fire(s + NBUF - 1, bufs[nf], sems[nf])
        for d in cps[cur]:
            d.wait()
        buf = bufs[cur]

        row_ctr = s * SB + lanes
        rowbase = lanes * K

        def e_body(e, accs, row_ctr=row_ctr, rowbase=rowbase, buf=buf):
            col = jnp.full((L,), e, jnp.int32)
            ctr = plsc.load_gather(ctr_rows, [row_ctr, col])
            new = []
            for k in range(K):
                v = plsc.load_gather(buf, [rowbase + k, col])
                new.append(accs[k] + ctr * v)
            return tuple(new)

        accs = lax.fori_loop(0, EMBED, e_body, (zero,) * K)
        for k in range(K):
            plsc.store_scatter(out_v, [rowbase + k], accs[k])

        pltpu.sync_copy(out_v, out_hbm.at[pl.ds((b0 + s * SB) * K, CTX_SB)])


def _body(ctr_idx_hbm, ctx_idx_hbm, ctr_tab, ctx_tab, out_hbm,
          ctr_idx_v, ctx_idx_v, ctr_rows, b0, b1, b2, b3, out_v,
          sem_c, s00, s01, s10, s11, s20, s21, s30, s31):
    _sc_body(ctr_idx_hbm, ctx_idx_hbm, ctr_tab, ctx_tab, out_hbm,
             ctr_idx_v, ctx_idx_v, ctr_rows, (b0, b1, b2, b3), out_v,
             sem_c, ((s00, s01), (s10, s11), (s20, s21), (s30, s31)))


_mesh = plsc.VectorSubcoreMesh(core_axis_name="c", subcore_axis_name="s")

_sc_kernel = functools.partial(
    pl.kernel,
    out_type=jax.ShapeDtypeStruct((B * K,), jnp.float32),
    mesh=_mesh,
    scratch_types=[
        pltpu.VMEM((BPW,), jnp.int32),
        pltpu.VMEM((BPW * K,), jnp.int32),
        pltpu.VMEM((BPW, EMBED), jnp.float32),
        pltpu.VMEM((CTX_SB, EMBED), jnp.float32),
        pltpu.VMEM((CTX_SB, EMBED), jnp.float32),
        pltpu.VMEM((CTX_SB, EMBED), jnp.float32),
        pltpu.VMEM((CTX_SB, EMBED), jnp.float32),
        pltpu.VMEM((CTX_SB,), jnp.float32),
        pltpu.SemaphoreType.DMA,
        pltpu.SemaphoreType.DMA,
        pltpu.SemaphoreType.DMA,
        pltpu.SemaphoreType.DMA,
        pltpu.SemaphoreType.DMA,
        pltpu.SemaphoreType.DMA,
        pltpu.SemaphoreType.DMA,
        pltpu.SemaphoreType.DMA,
        pltpu.SemaphoreType.DMA,
    ],
    compiler_params=pltpu.CompilerParams(
        use_tc_tiling_on_sc=False, needs_layout_passes=False),
)(_body)


def kernel(centers, context_negatives, center_table, context_table):
    ctr_idx = centers.reshape(B).astype(jnp.int32)
    ctx_idx = context_negatives.reshape(B * K).astype(jnp.int32)
    out = _sc_kernel(ctr_idx, ctx_idx, center_table, context_table)
    return out.reshape(B, K)


# speedup vs baseline: 2.2842x; 1.0057x over previous
import functools

import jax
import jax.numpy as jnp
from jax import lax
from jax.experimental import pallas as pl
from jax.experimental.pallas import tpu as pltpu
from jax.experimental.pallas import tpu_sc as plsc

VOCAB = 1000000
EMBED = 64
B = 16384
K = 20

NC = 2
NS = 16
L = 16
NW = NC * NS
BPW = B // NW
SB = 32
NSUB = BPW // SB
CTX_SB = SB * K


def _sc_body(ctr_idx_hbm, ctx_idx_hbm, ctr_tab, ctx_tab, out_hbm,
             ctr_idx_v, ctx_idx_v, ctr_rows, ctx_buf0, ctx_buf1, out_v,
             sem_c, sem0, sem1):
    wid = lax.axis_index("s") * NC + lax.axis_index("c")
    b0 = wid * BPW

    pltpu.sync_copy(ctr_idx_hbm.at[pl.ds(b0, BPW)], ctr_idx_v)
    pltpu.sync_copy(ctx_idx_hbm.at[pl.ds(b0 * K, BPW * K)], ctx_idx_v)

    ctr_cp = pltpu.async_copy(ctr_tab.at[ctr_idx_v], ctr_rows, sem_c)
    bufs = (ctx_buf0, ctx_buf1)
    sems = (sem0, sem1)
    cps = [None, None]
    cps[0] = pltpu.async_copy(
        ctx_tab.at[ctx_idx_v.at[pl.ds(0, CTX_SB)]], ctx_buf0, sem0)
    ctr_cp.wait()

    lanes = lax.iota(jnp.int32, L)
    zero = jnp.zeros((L,), jnp.float32)

    for s in range(NSUB):
        cur = s % 2
        if s + 1 < NSUB:
            nxt = (s + 1) % 2
            cps[nxt] = pltpu.async_copy(
                ctx_tab.at[ctx_idx_v.at[pl.ds((s + 1) * CTX_SB, CTX_SB)]],
                bufs[nxt], sems[nxt])
        cps[cur].wait()
        buf = bufs[cur]

        for g in range(SB // L):
            row_ctr = s * SB + g * L + lanes
            rowbase = (g * L + lanes) * K

            def e_body(e, accs, row_ctr=row_ctr, rowbase=rowbase, buf=buf):
                col = jnp.full((L,), e, jnp.int32)
                ctr = plsc.load_gather(ctr_rows, [row_ctr, col])
                new = []
                for k in range(K):
                    v = plsc.load_gather(buf, [rowbase + k, col])
                    new.append(accs[k] + ctr * v)
                return tuple(new)

            accs = lax.fori_loop(0, EMBED, e_body, (zero,) * K)
            for k in range(K):
                plsc.store_scatter(out_v, [rowbase + k], accs[k])

        pltpu.sync_copy(out_v, out_hbm.at[pl.ds((b0 + s * SB) * K, CTX_SB)])


_mesh = plsc.VectorSubcoreMesh(core_axis_name="c", subcore_axis_name="s")

_sc_kernel = functools.partial(
    pl.kernel,
    out_type=jax.ShapeDtypeStruct((B * K,), jnp.float32),
    mesh=_mesh,
    scratch_types=[
        pltpu.VMEM((BPW,), jnp.int32),
        pltpu.VMEM((BPW * K,), jnp.int32),
        pltpu.VMEM((BPW, EMBED), jnp.float32),
        pltpu.VMEM((CTX_SB, EMBED), jnp.float32),
        pltpu.VMEM((CTX_SB, EMBED), jnp.float32),
        pltpu.VMEM((CTX_SB,), jnp.float32),
        pltpu.SemaphoreType.DMA,
        pltpu.SemaphoreType.DMA,
        pltpu.SemaphoreType.DMA,
    ],
    compiler_params=pltpu.CompilerParams(
        use_tc_tiling_on_sc=False, needs_layout_passes=False),
)(_sc_body)


def kernel(centers, context_negatives, center_table, context_table):
    ctr_idx = centers.reshape(B).astype(jnp.int32)
    ctx_idx = context_negatives.reshape(B * K).astype(jnp.int32)
    out = _sc_kernel(ctr_idx, ctx_idx, center_table, context_table)
    return out.reshape(B, K)
